# baseline (speedup 1.0000x reference)
import functools

import jax
import jax.numpy as jnp
from jax import lax
from jax.experimental import pallas as pl
from jax.experimental.pallas import tpu as pltpu
from jax.experimental.pallas import tpu_sc as plsc

N = 10000
E = 320000
NR = 10240
HALF = NR // 2
ACCR = HALF + 128
RPS = ACCR // 16
WPS = HALF // 16
NS = 16
CHUNKS = 160
EP = NS * CHUNKS * 128
RB = 2048
GRID = NR // RB

_INT_MIN = -(2 ** 31)


def _pcall(body, **kw):
    return pl.pallas_call(body, **kw)


def _mm_body(x_ref, w_ref, o_ref):
    o_ref[...] = jnp.dot(x_ref[...], w_ref[...],
                         preferred_element_type=jnp.float32)


def _mm_gated_body(x_ref, g_ref, w_ref, o_ref):
    o_ref[...] = jnp.dot(x_ref[...] * g_ref[...], w_ref[...],
                         preferred_element_type=jnp.float32)


def _tc_mm(x, w):
    return _pcall(
        _mm_body,
        grid=(GRID,),
        in_specs=[pl.BlockSpec((RB, 128), lambda i: (i, 0)),
                  pl.BlockSpec((128, 128), lambda i: (0, 0))],
        out_specs=pl.BlockSpec((RB, 128), lambda i: (i, 0)),
        out_shape=jax.ShapeDtypeStruct((NR, 128), jnp.float32),
    )(x, w)


def _tc_mm_gated(x, g, w):
    return _pcall(
        _mm_gated_body,
        grid=(GRID,),
        in_specs=[pl.BlockSpec((RB, 128), lambda i: (i, 0)),
                  pl.BlockSpec((RB, 128), lambda i: (i, 0)),
                  pl.BlockSpec((128, 128), lambda i: (0, 0))],
        out_specs=pl.BlockSpec((RB, 128), lambda i: (i, 0)),
        out_shape=jax.ShapeDtypeStruct((NR, 128), jnp.float32),
    )(x, g, w)


def _scale_body(hw_ref, deg_ref, z_ref, dinv_ref):
    d = deg_ref[...]
    degv = lax.slice(d, (0, 0), (RB, 1)) + 1.0
    dinv = 1.0 / jnp.sqrt(degv)
    dinv_bc = jnp.broadcast_to(dinv, (RB, 128))
    dinv_ref[...] = dinv_bc
    z_ref[...] = dinv_bc * hw_ref[...]


def _tc_scale(hw, degp):
    return _pcall(
        _scale_body,
        grid=(GRID,),
        in_specs=[pl.BlockSpec((RB, 128), lambda i: (i, 0)),
                  pl.BlockSpec((RB, 128), lambda i: (i, 0))],
        out_specs=[pl.BlockSpec((RB, 128), lambda i: (i, 0)),
                   pl.BlockSpec((RB, 128), lambda i: (i, 0))],
        out_shape=[jax.ShapeDtypeStruct((NR, 128), jnp.float32),
                   jax.ShapeDtypeStruct((NR, 128), jnp.float32)],
    )(hw, degp)


def _post_body(s_ref, hw_ref, dinv_ref, b_ref, p_ref, h_ref, y_ref):
    s = s_ref[...]
    dinv = dinv_ref[...]
    agg = dinv * s + hw_ref[...] * dinv * dinv
    bb = b_ref[...]
    h = jnp.maximum(agg + bb[0:1, :], 0.0)
    h_ref[...] = h
    pvec = p_ref[...]
    rinv = 1.0 / jnp.sqrt(jnp.sum(pvec * pvec) * (1.0 / 128.0))
    y_ref[...] = jnp.dot(h, pvec, preferred_element_type=jnp.float32) * rinv


def _tc_post(s_full, hw, dinv, b2d, pbc):
    return _pcall(
        _post_body,
        grid=(GRID,),
        in_specs=[pl.BlockSpec((RB, 128), lambda i: (i, 0)),
                  pl.BlockSpec((RB, 128), lambda i: (i, 0)),
                  pl.BlockSpec((RB, 128), lambda i: (i, 0)),
                  pl.BlockSpec((8, 128), lambda i: (0, 0)),
                  pl.BlockSpec((128, 128), lambda i: (0, 0))],
        out_specs=[pl.BlockSpec((RB, 128), lambda i: (i, 0)),
                   pl.BlockSpec((RB, 128), lambda i: (i, 0))],
        out_shape=[jax.ShapeDtypeStruct((NR, 128), jnp.float32),
                   jax.ShapeDtypeStruct((NR, 128), jnp.float32)],
    )(s_full, hw, dinv, b2d, pbc)


def _topk_body(y_ref, mprev_ref, m_ref, g_ref, *, kk):
    y = y_ref[...] + 0.0
    sbits = lax.bitcast_convert_type(y, jnp.int32)
    v = sbits ^ (lax.shift_right_arithmetic(sbits, 31) & jnp.int32(0x7FFFFFFF))
    ri = lax.broadcasted_iota(jnp.int32, (80, 128), 0)
    ci = lax.broadcasted_iota(jnp.int32, (80, 128), 1)
    idx = ri * 128 + ci
    int_min = jnp.int32(_INT_MIN)
    valid = (mprev_ref[...] > 0.0) & (idx < N)
    v = jnp.where(valid, v, int_min)

    def tbody(t, tc):
        cand = tc ^ lax.shift_left(jnp.int32(1), jnp.int32(31) - t)
        cnt = jnp.sum((v >= cand).astype(jnp.int32))
        return jnp.where(cnt >= kk, cand, tc)
    tc = lax.fori_loop(0, 32, tbody, int_min)

    cnt_gt = jnp.sum((v > tc).astype(jnp.int32))
    r = jnp.int32(kk) - cnt_gt
    wt = jnp.where(v == tc, jnp.int32(16383) - idx, jnp.int32(-1))

    def t2body(t, t2):
        cand = t2 | lax.shift_left(jnp.int32(1), jnp.int32(13) - t)
        cnt = jnp.sum((wt >= cand).astype(jnp.int32))
        return jnp.where(cnt >= r, cand, t2)
    t2 = lax.fori_loop(0, 14, t2body, jnp.int32(0))

    sel = (v > tc) | ((v == tc) & (wt >= t2))
    m = sel.astype(jnp.float32)
    m_ref[...] = m
    g_ref[...] = m * jnp.tanh(y_ref[...])


def _tc_topk(y2d, mprev2d, kk):
    return _pcall(
        functools.partial(_topk_body, kk=kk),
        in_specs=[pl.BlockSpec((80, 128), lambda: (0, 0)),
                  pl.BlockSpec((80, 128), lambda: (0, 0))],
        out_specs=[pl.BlockSpec((80, 128), lambda: (0, 0)),
                   pl.BlockSpec((80, 128), lambda: (0, 0))],
        out_shape=[jax.ShapeDtypeStruct((80, 128), jnp.float32),
                   jax.ShapeDtypeStruct((80, 128), jnp.float32)],
    )(y2d, mprev2d)


def _pool_body(h_ref, m_ref, o_ref):
    @pl.when(pl.program_id(0) == 0)
    def _():
        o_ref[...] = jnp.zeros((8, 128), jnp.float32)
    ssum = jnp.sum(h_ref[...] * m_ref[...], axis=0, keepdims=True)
    o_ref[...] += jnp.broadcast_to(ssum, (8, 128))


def _tc_pool(h, mbc):
    return _pcall(
        _pool_body,
        grid=(GRID,),
        in_specs=[pl.BlockSpec((RB, 128), lambda i: (i, 0)),
                  pl.BlockSpec((RB, 128), lambda i: (i, 0))],
        out_specs=pl.BlockSpec((8, 128), lambda i: (0, 0)),
        out_shape=jax.ShapeDtypeStruct((8, 128), jnp.float32),
    )(h, mbc)


def _final_body(pool_ref, wd_ref, bd_ref, o_ref, *, lout):
    logits = jnp.dot(pool_ref[...], wd_ref[...],
                     preferred_element_type=jnp.float32) + bd_ref[...]
    lane = lax.broadcasted_iota(jnp.int32, (8, 128), 1)
    ok = lane < lout
    lm = jnp.where(ok, logits, jnp.float32(-1e30))
    mx = jnp.max(lm, axis=1, keepdims=True)
    e = jnp.where(ok, jnp.exp(lm - mx), 0.0)
    o_ref[...] = e / jnp.sum(e, axis=1, keepdims=True)


def _tc_final(pooled, wdp, bdp, lout):
    return _pcall(
        functools.partial(_final_body, lout=lout),
        in_specs=[pl.BlockSpec((8, 128), lambda: (0, 0)),
                  pl.BlockSpec((128, 128), lambda: (0, 0)),
                  pl.BlockSpec((8, 128), lambda: (0, 0))],
        out_specs=pl.BlockSpec((8, 128), lambda: (0, 0)),
        out_shape=jax.ShapeDtypeStruct((8, 128), jnp.float32),
    )(pooled, wdp, bdp)


def _dstmap_body(dst_ref, d0_ref, d1_ref):
    d = dst_ref[...]
    half = jnp.int32(HALF)
    d0_ref[...] = jnp.where(d < half, d, half)
    d1_ref[...] = jnp.where(d >= half, d - half, half)


def _tc_dstmap(dst2):
    return _pcall(
        _dstmap_body,
        in_specs=[pl.BlockSpec((EP // 128, 128), lambda: (0, 0))],
        out_specs=[pl.BlockSpec((EP // 128, 128), lambda: (0, 0)),
                   pl.BlockSpec((EP // 128, 128), lambda: (0, 0))],
        out_shape=[jax.ShapeDtypeStruct((EP // 128, 128), jnp.int32),
                   jax.ShapeDtypeStruct((EP // 128, 128), jnp.int32)],
    )(dst2)


def _sc_mesh():
    return plsc.VectorSubcoreMesh(core_axis_name="c", subcore_axis_name="s")


def _make_scatter():
    @functools.partial(
        pl.kernel,
        mesh=_sc_mesh(),
        out_type=jax.ShapeDtypeStruct((2, HALF, 128), jnp.float32),
        scratch_types=[
            pltpu.VMEM((CHUNKS, 128), jnp.int32),
            pltpu.VMEM((CHUNKS, 128), jnp.int32),
            pltpu.VMEM((128, 128), jnp.float32),
            pltpu.VMEM((128, 128), jnp.float32),
            pltpu.VMEM_SHARED((ACCR, 128), jnp.float32),
            pltpu.SemaphoreType.DMA,
            pltpu.SemaphoreType.DMA,
            pltpu.SemaphoreType.DMA,
            pltpu.SemaphoreType.DMA,
        ],
    )
    def k(src_hbm, dstl_hbm, tab_hbm, zero_hbm, out_hbm,
          src_idx, dst_idx, buf0, buf1, acc,
          gsem0, gsem1, ssem0, ssem1):
        c = lax.axis_index("c")
        s = lax.axis_index("s")
        pltpu.sync_copy(zero_hbm.at[pl.ds(s * RPS, RPS)],
                        acc.at[pl.ds(s * RPS, RPS)])
        pltpu.sync_copy(src_hbm.at[s], src_idx)
        pltpu.sync_copy(dstl_hbm.at[c].at[s], dst_idx)
        plsc.subcore_barrier()
        pairs = ((buf0, gsem0, ssem0), (buf1, gsem1, ssem1))
        pltpu.async_copy(tab_hbm.at[src_idx.at[0]], buf0, gsem0)

        def body(gg, carry):
            for b, (buf, gsem, ssem) in enumerate(pairs):
                g = 2 * gg + b
                obuf, ogsem, ossem = pairs[1 - b]
                pltpu.make_async_copy(tab_hbm.at[src_idx.at[g]], buf,
                                      gsem).wait()
                pltpu.async_copy(buf, acc.at[dst_idx.at[g]], ssem, add=True)

                @pl.when(g >= 1)
                def _():
                    pltpu.make_async_copy(obuf, acc.at[dst_idx.at[g - 1]],
                                          ossem).wait()

                @pl.when(g + 1 < CHUNKS)
                def _():
                    pltpu.async_copy(tab_hbm.at[src_idx.at[g + 1]],
                                     obuf, ogsem)
            return carry

        lax.fori_loop(0, CHUNKS // 2, body, 0)
        pltpu.make_async_copy(buf1, acc.at[dst_idx.at[CHUNKS - 1]],
                              ssem1).wait()
        plsc.subcore_barrier()
        pltpu.sync_copy(acc.at[pl.ds(s * WPS, WPS)],
                        out_hbm.at[c].at[pl.ds(s * WPS, WPS)])

    return k


def _b2d(b):
    return jnp.broadcast_to(b[None, :], (8, 128)).astype(jnp.float32)


def _pbc(p):
    return jnp.broadcast_to(p[:, None], (128, 128)).astype(jnp.float32)


def kernel(x, edge_index, i, W1, b1, p1, W2, b2, p2, W3, b3, Wd, bd):
    del i
    f32 = jnp.float32
    src = edge_index[0]
    dst = edge_index[1]
    pad_e = EP - E
    srcp = jnp.concatenate([src, jnp.full((pad_e,), N, jnp.int32)])
    dstp = jnp.concatenate([dst, jnp.full((pad_e,), N + 8, jnp.int32)])
    dl0, dl1 = _tc_dstmap(dstp.reshape(EP // 128, 128))
    dstl = jnp.stack([dl0, dl1]).reshape(2, NS, CHUNKS, 128)
    src3 = srcp.reshape(NS, CHUNKS, 128)
    xp = jnp.pad(x, ((0, NR - N), (0, 0)))
    zero128 = jnp.zeros((ACCR, 128), f32)
    ones2d = jnp.ones((80, 128), f32)

    rows_k = _make_scatter()

    def agg(tab):
        p = rows_k(src3, dstl, tab, zero128)
        return jnp.concatenate([p[0], p[1]], axis=0)

    def deg(m_bc):
        return agg(m_bc)

    k1 = (N + 1) // 2
    k2 = (k1 + 1) // 2

    deg1 = deg(jnp.ones((NR, 128), f32))
    hw1 = _tc_mm(xp, W1)
    z1, dinv1 = _tc_scale(hw1, deg1)
    h1, y1bc = _tc_post(agg(z1), hw1, dinv1, _b2d(b1), _pbc(p1))
    m1_2d, g1_2d = _tc_topk(y1bc[:, 0].reshape(80, 128), ones2d, k1)
    g1bc = jnp.broadcast_to(g1_2d.reshape(NR, 1), (NR, 128))

    deg2 = deg(jnp.broadcast_to(m1_2d.reshape(NR, 1), (NR, 128)))
    hw2 = _tc_mm_gated(h1, g1bc, W2)
    z2, dinv2 = _tc_scale(hw2, deg2)
    h2, y2bc = _tc_post(agg(z2), hw2, dinv2, _b2d(b2), _pbc(p2))
    m2_2d, g2_2d = _tc_topk(y2bc[:, 0].reshape(80, 128), m1_2d, k2)
    g2bc = jnp.broadcast_to(g2_2d.reshape(NR, 1), (NR, 128))

    deg3 = deg(jnp.broadcast_to(m2_2d.reshape(NR, 1), (NR, 128)))
    hw3 = _tc_mm_gated(h2, g2bc, W3)
    z3, dinv3 = _tc_scale(hw3, deg3)
    h3, _ = _tc_post(agg(z3), hw3, dinv3, _b2d(b3), _pbc(p1))

    m2bc = jnp.broadcast_to(m2_2d.reshape(NR, 1), (NR, 128))
    pooled = _tc_pool(h3, m2bc)
    lout = Wd.shape[1]
    wdp = jnp.zeros((128, 128), f32).at[:, :lout].set(Wd)
    bdp = jnp.zeros((8, 128), f32).at[:, :lout].set(
        jnp.broadcast_to(bd[None, :], (8, lout)))
    outp = _tc_final(pooled, wdp, bdp, lout)
    return outp[0:1, 0:lout]


# speedup vs baseline: 1.8840x; 1.8840x over previous
import functools

import jax
import jax.numpy as jnp
from jax import lax
from jax.experimental import pallas as pl
from jax.experimental.pallas import tpu as pltpu
from jax.experimental.pallas import tpu_sc as plsc

N = 10000
E = 320000
NR = 10240
HALF = NR // 2
ACCR = HALF + 128
RPS = ACCR // 16
WPS = HALF // 16
NS = 16
CHUNKS = 160
EP = NS * CHUNKS * 128
RB = 2048
GRID = NR // RB

_INT_MIN = -(2 ** 31)


def _pcall(body, **kw):
    return pl.pallas_call(body, **kw)


def _mm_body(x_ref, w_ref, o_ref):
    o_ref[...] = jnp.dot(x_ref[...], w_ref[...],
                         preferred_element_type=jnp.float32)


def _mm_gated_body(x_ref, g_ref, w_ref, o_ref):
    o_ref[...] = jnp.dot(x_ref[...] * g_ref[...], w_ref[...],
                         preferred_element_type=jnp.float32)


def _tc_mm(x, w):
    return _pcall(
        _mm_body,
        grid=(GRID,),
        in_specs=[pl.BlockSpec((RB, 128), lambda i: (i, 0)),
                  pl.BlockSpec((128, 128), lambda i: (0, 0))],
        out_specs=pl.BlockSpec((RB, 128), lambda i: (i, 0)),
        out_shape=jax.ShapeDtypeStruct((NR, 128), jnp.float32),
    )(x, w)


def _tc_mm_gated(x, g, w):
    return _pcall(
        _mm_gated_body,
        grid=(GRID,),
        in_specs=[pl.BlockSpec((RB, 128), lambda i: (i, 0)),
                  pl.BlockSpec((RB, 128), lambda i: (i, 0)),
                  pl.BlockSpec((128, 128), lambda i: (0, 0))],
        out_specs=pl.BlockSpec((RB, 128), lambda i: (i, 0)),
        out_shape=jax.ShapeDtypeStruct((NR, 128), jnp.float32),
    )(x, g, w)


def _scale_body(hw_ref, deg_ref, z_ref, dinv_ref):
    d = deg_ref[...]
    degv = lax.slice(d, (0, 0), (RB, 1)) + 1.0
    dinv = 1.0 / jnp.sqrt(degv)
    dinv_bc = jnp.broadcast_to(dinv, (RB, 128))
    dinv_ref[...] = dinv_bc
    z_ref[...] = dinv_bc * hw_ref[...]


def _tc_scale(hw, degp):
    return _pcall(
        _scale_body,
        grid=(GRID,),
        in_specs=[pl.BlockSpec((RB, 128), lambda i: (i, 0)),
                  pl.BlockSpec((RB, 16), lambda i: (i, 0))],
        out_specs=[pl.BlockSpec((RB, 128), lambda i: (i, 0)),
                   pl.BlockSpec((RB, 128), lambda i: (i, 0))],
        out_shape=[jax.ShapeDtypeStruct((NR, 128), jnp.float32),
                   jax.ShapeDtypeStruct((NR, 128), jnp.float32)],
    )(hw, degp)


def _post_body(s_ref, hw_ref, dinv_ref, b_ref, p_ref, h_ref, y_ref):
    s = s_ref[...]
    dinv = dinv_ref[...]
    agg = dinv * s + hw_ref[...] * dinv * dinv
    bb = b_ref[...]
    h = jnp.maximum(agg + bb[0:1, :], 0.0)
    h_ref[...] = h
    pvec = p_ref[...]
    rinv = 1.0 / jnp.sqrt(jnp.sum(pvec * pvec) * (1.0 / 128.0))
    y_ref[...] = jnp.dot(h, pvec, preferred_element_type=jnp.float32) * rinv


def _tc_post(s_full, hw, dinv, b2d, pbc):
    return _pcall(
        _post_body,
        grid=(GRID,),
        in_specs=[pl.BlockSpec((RB, 128), lambda i: (i, 0)),
                  pl.BlockSpec((RB, 128), lambda i: (i, 0)),
                  pl.BlockSpec((RB, 128), lambda i: (i, 0)),
                  pl.BlockSpec((8, 128), lambda i: (0, 0)),
                  pl.BlockSpec((128, 128), lambda i: (0, 0))],
        out_specs=[pl.BlockSpec((RB, 128), lambda i: (i, 0)),
                   pl.BlockSpec((RB, 128), lambda i: (i, 0))],
        out_shape=[jax.ShapeDtypeStruct((NR, 128), jnp.float32),
                   jax.ShapeDtypeStruct((NR, 128), jnp.float32)],
    )(s_full, hw, dinv, b2d, pbc)


def _topk_body(y_ref, mprev_ref, m_ref, g_ref, *, kk):
    y = y_ref[...] + 0.0
    sbits = lax.bitcast_convert_type(y, jnp.int32)
    v = sbits ^ (lax.shift_right_arithmetic(sbits, 31) & jnp.int32(0x7FFFFFFF))
    ri = lax.broadcasted_iota(jnp.int32, (80, 128), 0)
    ci = lax.broadcasted_iota(jnp.int32, (80, 128), 1)
    idx = ri * 128 + ci
    int_min = jnp.int32(_INT_MIN)
    valid = (mprev_ref[...] > 0.0) & (idx < N)
    v = jnp.where(valid, v, int_min)

    def tbody(t, tc):
        cand = tc ^ lax.shift_left(jnp.int32(1), jnp.int32(31) - t)
        cnt = jnp.sum((v >= cand).astype(jnp.int32))
        return jnp.where(cnt >= kk, cand, tc)
    tc = lax.fori_loop(0, 32, tbody, int_min)

    cnt_gt = jnp.sum((v > tc).astype(jnp.int32))
    r = jnp.int32(kk) - cnt_gt
    wt = jnp.where(v == tc, jnp.int32(16383) - idx, jnp.int32(-1))

    def t2body(t, t2):
        cand = t2 | lax.shift_left(jnp.int32(1), jnp.int32(13) - t)
        cnt = jnp.sum((wt >= cand).astype(jnp.int32))
        return jnp.where(cnt >= r, cand, t2)
    t2 = lax.fori_loop(0, 14, t2body, jnp.int32(0))

    sel = (v > tc) | ((v == tc) & (wt >= t2))
    m = sel.astype(jnp.float32)
    m_ref[...] = m
    g_ref[...] = m * jnp.tanh(y_ref[...])


def _tc_topk(y2d, mprev2d, kk):
    return _pcall(
        functools.partial(_topk_body, kk=kk),
        in_specs=[pl.BlockSpec((80, 128), lambda: (0, 0)),
                  pl.BlockSpec((80, 128), lambda: (0, 0))],
        out_specs=[pl.BlockSpec((80, 128), lambda: (0, 0)),
                   pl.BlockSpec((80, 128), lambda: (0, 0))],
        out_shape=[jax.ShapeDtypeStruct((80, 128), jnp.float32),
                   jax.ShapeDtypeStruct((80, 128), jnp.float32)],
    )(y2d, mprev2d)


def _pool_body(h_ref, m_ref, o_ref):
    @pl.when(pl.program_id(0) == 0)
    def _():
        o_ref[...] = jnp.zeros((8, 128), jnp.float32)
    ssum = jnp.sum(h_ref[...] * m_ref[...], axis=0, keepdims=True)
    o_ref[...] += jnp.broadcast_to(ssum, (8, 128))


def _tc_pool(h, mbc):
    return _pcall(
        _pool_body,
        grid=(GRID,),
        in_specs=[pl.BlockSpec((RB, 128), lambda i: (i, 0)),
                  pl.BlockSpec((RB, 128), lambda i: (i, 0))],
        out_specs=pl.BlockSpec((8, 128), lambda i: (0, 0)),
        out_shape=jax.ShapeDtypeStruct((8, 128), jnp.float32),
    )(h, mbc)


def _final_body(pool_ref, wd_ref, bd_ref, o_ref, *, lout):
    logits = jnp.dot(pool_ref[...], wd_ref[...],
                     preferred_element_type=jnp.float32) + bd_ref[...]
    lane = lax.broadcasted_iota(jnp.int32, (8, 128), 1)
    ok = lane < lout
    lm = jnp.where(ok, logits, jnp.float32(-1e30))
    mx = jnp.max(lm, axis=1, keepdims=True)
    e = jnp.where(ok, jnp.exp(lm - mx), 0.0)
    o_ref[...] = e / jnp.sum(e, axis=1, keepdims=True)


def _tc_final(pooled, wdp, bdp, lout):
    return _pcall(
        functools.partial(_final_body, lout=lout),
        in_specs=[pl.BlockSpec((8, 128), lambda: (0, 0)),
                  pl.BlockSpec((128, 128), lambda: (0, 0)),
                  pl.BlockSpec((8, 128), lambda: (0, 0))],
        out_specs=pl.BlockSpec((8, 128), lambda: (0, 0)),
        out_shape=jax.ShapeDtypeStruct((8, 128), jnp.float32),
    )(pooled, wdp, bdp)


def _dstmap_body(dst_ref, d0_ref, d1_ref):
    d = dst_ref[...]
    half = jnp.int32(HALF)
    d0_ref[...] = jnp.where(d < half, d, half)
    d1_ref[...] = jnp.where(d >= half, d - half, half)


def _tc_dstmap(dst2):
    return _pcall(
        _dstmap_body,
        in_specs=[pl.BlockSpec((EP // 128, 128), lambda: (0, 0))],
        out_specs=[pl.BlockSpec((EP // 128, 128), lambda: (0, 0)),
                   pl.BlockSpec((EP // 128, 128), lambda: (0, 0))],
        out_shape=[jax.ShapeDtypeStruct((EP // 128, 128), jnp.int32),
                   jax.ShapeDtypeStruct((EP // 128, 128), jnp.int32)],
    )(dst2)


def _sc_mesh():
    return plsc.VectorSubcoreMesh(core_axis_name="c", subcore_axis_name="s")


def _make_scatter():
    @functools.partial(
        pl.kernel,
        mesh=_sc_mesh(),
        out_type=jax.ShapeDtypeStruct((2, HALF, 128), jnp.float32),
        scratch_types=[
            pltpu.VMEM((CHUNKS, 128), jnp.int32),
            pltpu.VMEM((CHUNKS, 128), jnp.int32),
            pltpu.VMEM((128, 128), jnp.float32),
            pltpu.VMEM((128, 128), jnp.float32),
            pltpu.VMEM_SHARED((ACCR, 128), jnp.float32),
            pltpu.SemaphoreType.DMA,
            pltpu.SemaphoreType.DMA,
            pltpu.SemaphoreType.DMA,
            pltpu.SemaphoreType.DMA,
        ],
    )
    def k(src_hbm, dstl_hbm, tab_hbm, zero_hbm, out_hbm,
          src_idx, dst_idx, buf0, buf1, acc,
          gsem0, gsem1, ssem0, ssem1):
        c = lax.axis_index("c")
        s = lax.axis_index("s")
        pltpu.sync_copy(zero_hbm.at[pl.ds(s * RPS, RPS)],
                        acc.at[pl.ds(s * RPS, RPS)])
        pltpu.sync_copy(src_hbm.at[s], src_idx)
        pltpu.sync_copy(dstl_hbm.at[c].at[s], dst_idx)
        plsc.subcore_barrier()
        pairs = ((buf0, gsem0, ssem0), (buf1, gsem1, ssem1))
        pltpu.async_copy(tab_hbm.at[src_idx.at[0]], buf0, gsem0)

        def body(gg, carry):
            for b, (buf, gsem, ssem) in enumerate(pairs):
                g = 2 * gg + b
                obuf, ogsem, ossem = pairs[1 - b]
                pltpu.make_async_copy(tab_hbm.at[src_idx.at[g]], buf,
                                      gsem).wait()
                pltpu.async_copy(buf, acc.at[dst_idx.at[g]], ssem, add=True)

                @pl.when(g >= 1)
                def _():
                    pltpu.make_async_copy(obuf, acc.at[dst_idx.at[g - 1]],
                                          ossem).wait()

                @pl.when(g + 1 < CHUNKS)
                def _():
                    pltpu.async_copy(tab_hbm.at[src_idx.at[g + 1]],
                                     obuf, ogsem)
            return carry

        lax.fori_loop(0, CHUNKS // 2, body, 0)
        pltpu.make_async_copy(buf1, acc.at[dst_idx.at[CHUNKS - 1]],
                              ssem1).wait()
        plsc.subcore_barrier()
        pltpu.sync_copy(acc.at[pl.ds(s * WPS, WPS)],
                        out_hbm.at[c].at[pl.ds(s * WPS, WPS)])

    return k


def _make_deghist():
    @functools.partial(
        pl.kernel,
        mesh=_sc_mesh(),
        compiler_params=pltpu.CompilerParams(needs_layout_passes=False),
        out_type=jax.ShapeDtypeStruct((32, NR), jnp.float32),
        scratch_types=[
            pltpu.VMEM((EP // 32 // 128, 128), jnp.int32),
            pltpu.VMEM((EP // 32 // 128, 128), jnp.int32),
            pltpu.VMEM((NR,), jnp.float32),
            pltpu.VMEM((NR,), jnp.float32),
        ],
    )
    def k(src_hbm, dst_hbm, m_hbm, zero_hbm, out_hbm,
          src_idx, dst_idx, m_tile, hist):
        c = lax.axis_index("c")
        s = lax.axis_index("s")
        wid = s * 2 + c
        pltpu.sync_copy(zero_hbm, hist)
        pltpu.sync_copy(m_hbm, m_tile)
        pltpu.sync_copy(src_hbm.at[wid], src_idx)
        pltpu.sync_copy(dst_hbm.at[wid], dst_idx)

        def body(g, carry):
            for j in range(8):
                sv = src_idx[g, pl.ds(16 * j, 16)]
                dv = dst_idx[g, pl.ds(16 * j, 16)]
                mv = plsc.load_gather(m_tile, [sv])
                plsc.addupdate_scatter(hist, [dv], mv)
            return carry

        lax.fori_loop(0, EP // 32 // 128, body, 0)
        pltpu.sync_copy(hist, out_hbm.at[wid])

    return k


def _degsum_body(h_ref, o_ref):
    d = jnp.sum(h_ref[...], axis=0, keepdims=True)
    o_ref[...] = jnp.broadcast_to(d, (8, RB))


def _tc_degsum(hists):
    return _pcall(
        _degsum_body,
        grid=(GRID,),
        in_specs=[pl.BlockSpec((32, RB), lambda i: (0, i))],
        out_specs=pl.BlockSpec((8, RB), lambda i: (0, i)),
        out_shape=jax.ShapeDtypeStruct((8, NR), jnp.float32),
    )(hists)


def _b2d(b):
    return jnp.broadcast_to(b[None, :], (8, 128)).astype(jnp.float32)


def _pbc(p):
    return jnp.broadcast_to(p[:, None], (128, 128)).astype(jnp.float32)


def kernel(x, edge_index, i, W1, b1, p1, W2, b2, p2, W3, b3, Wd, bd):
    del i
    f32 = jnp.float32
    src = edge_index[0]
    dst = edge_index[1]
    pad_e = EP - E
    srcp = jnp.concatenate([src, jnp.full((pad_e,), N, jnp.int32)])
    dstp = jnp.concatenate([dst, jnp.full((pad_e,), N + 8, jnp.int32)])
    dl0, dl1 = _tc_dstmap(dstp.reshape(EP // 128, 128))
    dstl = jnp.stack([dl0, dl1]).reshape(2, NS, CHUNKS, 128)
    src3 = srcp.reshape(NS, CHUNKS, 128)
    xp = jnp.pad(x, ((0, NR - N), (0, 0)))
    zero128 = jnp.zeros((ACCR, 128), f32)
    ones2d = jnp.ones((80, 128), f32)

    src32 = srcp.reshape(32, EP // 32 // 128, 128)
    dst32 = dstp.reshape(32, EP // 32 // 128, 128)
    zero_nr = jnp.zeros((NR,), f32)

    rows_k = _make_scatter()
    deghist_k = _make_deghist()

    def agg(tab):
        p = rows_k(src3, dstl, tab, zero128)
        return jnp.concatenate([p[0], p[1]], axis=0)

    def deg(m_flat):
        hists = deghist_k(src32, dst32, m_flat, zero_nr)
        dsum = _tc_degsum(hists)
        return jnp.broadcast_to(dsum[0].reshape(NR, 1), (NR, 16))

    k1 = (N + 1) // 2
    k2 = (k1 + 1) // 2

    deg1 = deg(jnp.ones((NR,), f32))
    hw1 = _tc_mm(xp, W1)
    z1, dinv1 = _tc_scale(hw1, deg1)
    h1, y1bc = _tc_post(agg(z1), hw1, dinv1, _b2d(b1), _pbc(p1))
    m1_2d, g1_2d = _tc_topk(y1bc[:, 0].reshape(80, 128), ones2d, k1)
    g1bc = jnp.broadcast_to(g1_2d.reshape(NR, 1), (NR, 128))

    deg2 = deg(m1_2d.reshape(NR))
    hw2 = _tc_mm_gated(h1, g1bc, W2)
    z2, dinv2 = _tc_scale(hw2, deg2)
    h2, y2bc = _tc_post(agg(z2), hw2, dinv2, _b2d(b2), _pbc(p2))
    m2_2d, g2_2d = _tc_topk(y2bc[:, 0].reshape(80, 128), m1_2d, k2)
    g2bc = jnp.broadcast_to(g2_2d.reshape(NR, 1), (NR, 128))

    deg3 = deg(m2_2d.reshape(NR))
    hw3 = _tc_mm_gated(h2, g2bc, W3)
    z3, dinv3 = _tc_scale(hw3, deg3)
    h3, _ = _tc_post(agg(z3), hw3, dinv3, _b2d(b3), _pbc(p1))

    m2bc = jnp.broadcast_to(m2_2d.reshape(NR, 1), (NR, 128))
    pooled = _tc_pool(h3, m2bc)
    lout = Wd.shape[1]
    wdp = jnp.zeros((128, 128), f32).at[:, :lout].set(Wd)
    bdp = jnp.zeros((8, 128), f32).at[:, :lout].set(
        jnp.broadcast_to(bd[None, :], (8, lout)))
    outp = _tc_final(pooled, wdp, bdp, lout)
    return outp[0:1, 0:lout]


# speedup vs baseline: 2.7570x; 1.4634x over previous
import functools

import jax
import jax.numpy as jnp
from jax import lax
from jax.experimental import pallas as pl
from jax.experimental.pallas import tpu as pltpu
from jax.experimental.pallas import tpu_sc as plsc

N = 10000
E = 320000
NR = 10240
HALF = NR // 2
ACCR = HALF + 128
RPS = ACCR // 16
WPS = HALF // 16
NS = 16
CHUNKS = 160
EP = NS * CHUNKS * 128
EPT = EP // 32
RB = 2048
GRID = NR // RB

_INT_MIN = -(2 ** 31)


def _pcall(body, **kw):
    return pl.pallas_call(body, **kw)


def _mm_body(x_ref, w_ref, o_ref):
    o_ref[...] = jnp.dot(x_ref[...], w_ref[...],
                         preferred_element_type=jnp.float32)


def _mm_gated_body(x_ref, g_ref, w_ref, o_ref):
    o_ref[...] = jnp.dot(x_ref[...] * g_ref[...], w_ref[...],
                         preferred_element_type=jnp.float32)


def _tc_mm(x, w):
    return _pcall(
        _mm_body,
        grid=(GRID,),
        in_specs=[pl.BlockSpec((RB, 128), lambda i: (i, 0)),
                  pl.BlockSpec((128, 128), lambda i: (0, 0))],
        out_specs=pl.BlockSpec((RB, 128), lambda i: (i, 0)),
        out_shape=jax.ShapeDtypeStruct((NR, 128), jnp.float32),
    )(x, w)


def _tc_mm_gated(x, g, w):
    return _pcall(
        _mm_gated_body,
        grid=(GRID,),
        in_specs=[pl.BlockSpec((RB, 128), lambda i: (i, 0)),
                  pl.BlockSpec((RB, 128), lambda i: (i, 0)),
                  pl.BlockSpec((128, 128), lambda i: (0, 0))],
        out_specs=pl.BlockSpec((RB, 128), lambda i: (i, 0)),
        out_shape=jax.ShapeDtypeStruct((NR, 128), jnp.float32),
    )(x, g, w)


def _scale_body(hw_ref, deg_ref, z_ref, dinv_ref):
    d = deg_ref[...]
    degv = lax.slice(d, (0, 0), (RB, 1)) + 1.0
    dinv = 1.0 / jnp.sqrt(degv)
    dinv_bc = jnp.broadcast_to(dinv, (RB, 128))
    dinv_ref[...] = dinv_bc
    z_ref[...] = dinv_bc * hw_ref[...]


def _tc_scale(hw, degp):
    return _pcall(
        _scale_body,
        grid=(GRID,),
        in_specs=[pl.BlockSpec((RB, 128), lambda i: (i, 0)),
                  pl.BlockSpec((RB, 16), lambda i: (i, 0))],
        out_specs=[pl.BlockSpec((RB, 128), lambda i: (i, 0)),
                   pl.BlockSpec((RB, 128), lambda i: (i, 0))],
        out_shape=[jax.ShapeDtypeStruct((NR, 128), jnp.float32),
                   jax.ShapeDtypeStruct((NR, 128), jnp.float32)],
    )(hw, degp)


def _post_body(parts_ref, hw_ref, dinv_ref, b_ref, p_ref, h_ref, y_ref):
    pp_ = parts_ref[...]
    s = pp_[0] + pp_[1]
    dinv = dinv_ref[...]
    agg = dinv * s + hw_ref[...] * dinv * dinv
    bb = b_ref[...]
    h = jnp.maximum(agg + bb[0:1, :], 0.0)
    h_ref[...] = h
    pvec = p_ref[...]
    rinv = 1.0 / jnp.sqrt(jnp.sum(pvec * pvec) * (1.0 / 128.0))
    y_ref[...] = jnp.dot(h, pvec, preferred_element_type=jnp.float32) * rinv


def _tc_post(parts, hw, dinv, b2d, pbc):
    return _pcall(
        _post_body,
        grid=(GRID,),
        in_specs=[pl.BlockSpec((2, RB, 128), lambda i: (0, i, 0)),
                  pl.BlockSpec((RB, 128), lambda i: (i, 0)),
                  pl.BlockSpec((RB, 128), lambda i: (i, 0)),
                  pl.BlockSpec((8, 128), lambda i: (0, 0)),
                  pl.BlockSpec((128, 128), lambda i: (0, 0))],
        out_specs=[pl.BlockSpec((RB, 128), lambda i: (i, 0)),
                   pl.BlockSpec((RB, 128), lambda i: (i, 0))],
        out_shape=[jax.ShapeDtypeStruct((NR, 128), jnp.float32),
                   jax.ShapeDtypeStruct((NR, 128), jnp.float32)],
    )(parts, hw, dinv, b2d, pbc)


def _topk_body(y_ref, mprev_ref, m_ref, g_ref, *, kk):
    y = y_ref[...] + 0.0
    sbits = lax.bitcast_convert_type(y, jnp.int32)
    v = sbits ^ (lax.shift_right_arithmetic(sbits, 31) & jnp.int32(0x7FFFFFFF))
    ri = lax.broadcasted_iota(jnp.int32, (80, 128), 0)
    ci = lax.broadcasted_iota(jnp.int32, (80, 128), 1)
    idx = ri * 128 + ci
    int_min = jnp.int32(_INT_MIN)
    valid = (mprev_ref[...] > 0.0) & (idx < N)
    v = jnp.where(valid, v, int_min)

    def tbody(t, tc):
        cand = tc ^ lax.shift_left(jnp.int32(1), jnp.int32(31) - t)
        cnt = jnp.sum((v >= cand).astype(jnp.int32))
        return jnp.where(cnt >= kk, cand, tc)
    tc = lax.fori_loop(0, 32, tbody, int_min)

    cnt_gt = jnp.sum((v > tc).astype(jnp.int32))
    r = jnp.int32(kk) - cnt_gt
    wt = jnp.where(v == tc, jnp.int32(16383) - idx, jnp.int32(-1))

    def t2body(t, t2):
        cand = t2 | lax.shift_left(jnp.int32(1), jnp.int32(13) - t)
        cnt = jnp.sum((wt >= cand).astype(jnp.int32))
        return jnp.where(cnt >= r, cand, t2)
    t2 = lax.fori_loop(0, 14, t2body, jnp.int32(0))

    sel = (v > tc) | ((v == tc) & (wt >= t2))
    m = sel.astype(jnp.float32)
    m_ref[...] = m
    g_ref[...] = m * jnp.tanh(y_ref[...])


def _tc_topk(y2d, mprev2d, kk):
    return _pcall(
        functools.partial(_topk_body, kk=kk),
        in_specs=[pl.BlockSpec((80, 128), lambda: (0, 0)),
                  pl.BlockSpec((80, 128), lambda: (0, 0))],
        out_specs=[pl.BlockSpec((80, 128), lambda: (0, 0)),
                   pl.BlockSpec((80, 128), lambda: (0, 0))],
        out_shape=[jax.ShapeDtypeStruct((80, 128), jnp.float32),
                   jax.ShapeDtypeStruct((80, 128), jnp.float32)],
    )(y2d, mprev2d)


def _pool_body(h_ref, m_ref, o_ref):
    @pl.when(pl.program_id(0) == 0)
    def _():
        o_ref[...] = jnp.zeros((8, 128), jnp.float32)
    ssum = jnp.sum(h_ref[...] * m_ref[...], axis=0, keepdims=True)
    o_ref[...] += jnp.broadcast_to(ssum, (8, 128))


def _tc_pool(h, mbc):
    return _pcall(
        _pool_body,
        grid=(GRID,),
        in_specs=[pl.BlockSpec((RB, 128), lambda i: (i, 0)),
                  pl.BlockSpec((RB, 128), lambda i: (i, 0))],
        out_specs=pl.BlockSpec((8, 128), lambda i: (0, 0)),
        out_shape=jax.ShapeDtypeStruct((8, 128), jnp.float32),
    )(h, mbc)


def _final_body(pool_ref, wd_ref, bd_ref, o_ref, *, lout):
    logits = jnp.dot(pool_ref[...], wd_ref[...],
                     preferred_element_type=jnp.float32) + bd_ref[...]
    lane = lax.broadcasted_iota(jnp.int32, (8, 128), 1)
    ok = lane < lout
    lm = jnp.where(ok, logits, jnp.float32(-1e30))
    mx = jnp.max(lm, axis=1, keepdims=True)
    e = jnp.where(ok, jnp.exp(lm - mx), 0.0)
    o_ref[...] = e / jnp.sum(e, axis=1, keepdims=True)


def _tc_final(pooled, wdp, bdp, lout):
    return _pcall(
        functools.partial(_final_body, lout=lout),
        in_specs=[pl.BlockSpec((8, 128), lambda: (0, 0)),
                  pl.BlockSpec((128, 128), lambda: (0, 0)),
                  pl.BlockSpec((8, 128), lambda: (0, 0))],
        out_specs=pl.BlockSpec((8, 128), lambda: (0, 0)),
        out_shape=jax.ShapeDtypeStruct((8, 128), jnp.float32),
    )(pooled, wdp, bdp)


def _dstmap_body(dst_ref, d0_ref, d1_ref):
    d = dst_ref[...]
    half = jnp.int32(HALF)
    d0_ref[...] = jnp.where(d < half, d, half)
    d1_ref[...] = jnp.where(d >= half, d - half, half)


def _tc_dstmap(dst2):
    return _pcall(
        _dstmap_body,
        in_specs=[pl.BlockSpec((EP // 128, 128), lambda: (0, 0))],
        out_specs=[pl.BlockSpec((EP // 128, 128), lambda: (0, 0)),
                   pl.BlockSpec((EP // 128, 128), lambda: (0, 0))],
        out_shape=[jax.ShapeDtypeStruct((EP // 128, 128), jnp.int32),
                   jax.ShapeDtypeStruct((EP // 128, 128), jnp.int32)],
    )(dst2)


def _sc_mesh():
    return plsc.VectorSubcoreMesh(core_axis_name="c", subcore_axis_name="s")


def _make_scatter():
    @functools.partial(
        pl.kernel,
        mesh=_sc_mesh(),
        out_type=jax.ShapeDtypeStruct((2, NR, 128), jnp.float32),
        scratch_types=[
            pltpu.VMEM((EPT // 128, 128), jnp.int32),
            pltpu.VMEM((EPT // 128, 128), jnp.int32),
            pltpu.VMEM((128, 128), jnp.float32),
            pltpu.VMEM_SHARED((NR, 128), jnp.float32),
            pltpu.SemaphoreType.DMA,
            pltpu.SemaphoreType.DMA,
        ],
    )
    def k(src_hbm, dst_hbm, tab_hbm, zero_hbm, out_hbm,
          src_idx, dst_idx, buf, acc, gsem, ssem):
        c = lax.axis_index("c")
        s = lax.axis_index("s")
        wid = s * 2 + c
        row0 = s * (NR // 16)
        pltpu.sync_copy(zero_hbm.at[pl.ds(row0, NR // 16)],
                        acc.at[pl.ds(row0, NR // 16)])
        pltpu.sync_copy(src_hbm.at[wid], src_idx)
        pltpu.sync_copy(dst_hbm.at[wid], dst_idx)
        plsc.subcore_barrier()
        pltpu.async_copy(tab_hbm.at[src_idx.at[0]], buf, gsem)

        nch = EPT // 128

        def body(g, carry):
            pltpu.make_async_copy(tab_hbm.at[src_idx.at[g]], buf,
                                  gsem).wait()
            pltpu.async_copy(buf, acc.at[dst_idx.at[g]], ssem, add=True)
            pltpu.make_async_copy(buf, acc.at[dst_idx.at[g]], ssem).wait()

            @pl.when(g + 1 < nch)
            def _():
                pltpu.async_copy(tab_hbm.at[src_idx.at[g + 1]], buf, gsem)
            return carry

        lax.fori_loop(0, nch, body, 0)
        plsc.subcore_barrier()
        pltpu.sync_copy(acc.at[pl.ds(row0, NR // 16)],
                        out_hbm.at[c].at[pl.ds(row0, NR // 16)])

    return k


def _make_deghist():
    @functools.partial(
        pl.kernel,
        mesh=_sc_mesh(),
        compiler_params=pltpu.CompilerParams(needs_layout_passes=False),
        out_type=jax.ShapeDtypeStruct((32, NR), jnp.float32),
        scratch_types=[
            pltpu.VMEM((EP // 32 // 128, 128), jnp.int32),
            pltpu.VMEM((EP // 32 // 128, 128), jnp.int32),
            pltpu.VMEM((NR,), jnp.float32),
            pltpu.VMEM((NR,), jnp.float32),
        ],
    )
    def k(src_hbm, dst_hbm, m_hbm, zero_hbm, out_hbm,
          src_idx, dst_idx, m_tile, hist):
        c = lax.axis_index("c")
        s = lax.axis_index("s")
        wid = s * 2 + c
        pltpu.sync_copy(zero_hbm, hist)
        pltpu.sync_copy(m_hbm, m_tile)
        pltpu.sync_copy(src_hbm.at[wid], src_idx)
        pltpu.sync_copy(dst_hbm.at[wid], dst_idx)

        def body(g, carry):
            for j in range(8):
                sv = src_idx[g, pl.ds(16 * j, 16)]
                dv = dst_idx[g, pl.ds(16 * j, 16)]
                mv = plsc.load_gather(m_tile, [sv])
                plsc.addupdate_scatter(hist, [dv], mv)
            return carry

        lax.fori_loop(0, EP // 32 // 128, body, 0)
        pltpu.sync_copy(hist, out_hbm.at[wid])

    return k


def _degsum_body(h_ref, o_ref):
    d = jnp.sum(h_ref[...], axis=0, keepdims=True)
    o_ref[...] = jnp.broadcast_to(d, (8, RB))


def _tc_degsum(hists):
    return _pcall(
        _degsum_body,
        grid=(GRID,),
        in_specs=[pl.BlockSpec((32, RB), lambda i: (0, i))],
        out_specs=pl.BlockSpec((8, RB), lambda i: (0, i)),
        out_shape=jax.ShapeDtypeStruct((8, NR), jnp.float32),
    )(hists)


def _b2d(b):
    return jnp.broadcast_to(b[None, :], (8, 128)).astype(jnp.float32)


def _pbc(p):
    return jnp.broadcast_to(p[:, None], (128, 128)).astype(jnp.float32)


def kernel(x, edge_index, i, W1, b1, p1, W2, b2, p2, W3, b3, Wd, bd):
    del i
    f32 = jnp.float32
    src = edge_index[0]
    dst = edge_index[1]
    pad_e = EP - E
    srcp = jnp.concatenate([src, jnp.full((pad_e,), N, jnp.int32)])
    dstp = jnp.concatenate([dst, jnp.full((pad_e,), N + 8, jnp.int32)])
    xp = jnp.pad(x, ((0, NR - N), (0, 0)))
    zero_nr128 = jnp.zeros((NR, 128), f32)
    ones2d = jnp.ones((80, 128), f32)

    src32 = srcp.reshape(32, EP // 32 // 128, 128)
    dst32 = dstp.reshape(32, EP // 32 // 128, 128)
    zero_nr = jnp.zeros((NR,), f32)

    rows_k = _make_scatter()
    deghist_k = _make_deghist()

    def agg(tab):
        return rows_k(src32, dst32, tab, zero_nr128)

    def deg(m_flat):
        hists = deghist_k(src32, dst32, m_flat, zero_nr)
        dsum = _tc_degsum(hists)
        return jnp.broadcast_to(dsum[0].reshape(NR, 1), (NR, 16))

    k1 = (N + 1) // 2
    k2 = (k1 + 1) // 2

    deg1 = deg(jnp.ones((NR,), f32))
    hw1 = _tc_mm(xp, W1)
    z1, dinv1 = _tc_scale(hw1, deg1)
    h1, y1bc = _tc_post(agg(z1), hw1, dinv1, _b2d(b1), _pbc(p1))
    m1_2d, g1_2d = _tc_topk(y1bc[:, 0].reshape(80, 128), ones2d, k1)
    g1bc = jnp.broadcast_to(g1_2d.reshape(NR, 1), (NR, 128))

    deg2 = deg(m1_2d.reshape(NR))
    hw2 = _tc_mm_gated(h1, g1bc, W2)
    z2, dinv2 = _tc_scale(hw2, deg2)
    h2, y2bc = _tc_post(agg(z2), hw2, dinv2, _b2d(b2), _pbc(p2))
    m2_2d, g2_2d = _tc_topk(y2bc[:, 0].reshape(80, 128), m1_2d, k2)
    g2bc = jnp.broadcast_to(g2_2d.reshape(NR, 1), (NR, 128))

    deg3 = deg(m2_2d.reshape(NR))
    hw3 = _tc_mm_gated(h2, g2bc, W3)
    z3, dinv3 = _tc_scale(hw3, deg3)
    h3, _ = _tc_post(agg(z3), hw3, dinv3, _b2d(b3), _pbc(p1))

    m2bc = jnp.broadcast_to(m2_2d.reshape(NR, 1), (NR, 128))
    pooled = _tc_pool(h3, m2bc)
    lout = Wd.shape[1]
    wdp = jnp.zeros((128, 128), f32).at[:, :lout].set(Wd)
    bdp = jnp.zeros((8, 128), f32).at[:, :lout].set(
        jnp.broadcast_to(bd[None, :], (8, lout)))
    outp = _tc_final(pooled, wdp, bdp, lout)
    return outp[0:1, 0:lout]
